# baseline (speedup 1.0000x reference)
import functools
import jax
import jax.numpy as jnp
from jax import lax
from jax.experimental import pallas as pl
from jax.experimental.pallas import tpu as pltpu
from jax.experimental.pallas import tpu_sc as plsc

_ALPHA = 4
_NUM_CORES = 2
_NUM_SUBCORES = 16
_NBUF = 4
_HSPLIT = 2


def _make_sc_gather(B, C, T, S, H, W):
    n_slow = B * C * S
    n_workers = _NUM_CORES * _NUM_SUBCORES
    per_w = n_slow // n_workers
    assert per_w * n_workers == n_slow and H % _HSPLIT == 0
    hh = H // _HSPLIT
    n_chunks = per_w * _HSPLIT
    mesh = plsc.VectorSubcoreMesh(core_axis_name="c", subcore_axis_name="s")

    @functools.partial(
        pl.kernel,
        mesh=mesh,
        out_type=jax.ShapeDtypeStruct((B, C, S, H, W), jnp.float32),
        scratch_types=[
            pltpu.VMEM((_NBUF, hh, W), jnp.float32),
            pltpu.SemaphoreType.DMA((_NBUF,)),
            pltpu.SemaphoreType.DMA((_NBUF,)),
        ],
    )
    def sc_gather(x_hbm, slow_hbm, buf, sem_in, sem_out):
        wid = lax.axis_index("s") * _NUM_CORES + lax.axis_index("c")

        def job(m):
            r = wid * per_w + m // _HSPLIT
            h0 = (m % _HSPLIT) * hh
            i = r // S
            k = r - i * S
            t = ((T - 1) * k) // (S - 1)
            b = i // C
            c = i - b * C
            return (b, c, t, h0), (b, c, k, h0)

        def start_in(m):
            (b, c, t, h0), _ = job(m)
            return pltpu.async_copy(
                x_hbm.at[b, c, t, pl.ds(h0, hh)], buf.at[m % _NBUF],
                sem_in.at[m % _NBUF])

        def start_out(m):
            _, (b, c, k, h0) = job(m)
            return pltpu.async_copy(
                buf.at[m % _NBUF], slow_hbm.at[b, c, k, pl.ds(h0, hh)],
                sem_out.at[m % _NBUF])

        in_h = [None] * n_chunks
        out_h = [None] * n_chunks
        for m in range(n_chunks):
            if m >= _NBUF:
                out_h[m - _NBUF].wait()
            in_h[m] = start_in(m)
            if m >= 1:
                in_h[m - 1].wait()
                out_h[m - 1] = start_out(m - 1)
        in_h[n_chunks - 1].wait()
        out_h[n_chunks - 1] = start_out(n_chunks - 1)
        for j in range(max(n_chunks - _NBUF, 0), n_chunks):
            if out_h[j] is not None and j >= n_chunks - _NBUF:
                out_h[j].wait()

    return sc_gather


def kernel(frames):
    squeeze = frames.ndim == 4
    x = frames[None] if squeeze else frames
    B, C, T, H, W = x.shape
    S = T // _ALPHA
    slow = _make_sc_gather(B, C, T, S, H, W)(x)
    if squeeze:
        slow = slow[0]
    return (slow, frames)


# speedup vs baseline: 1.1610x; 1.1610x over previous
import numpy as np
import jax
import jax.numpy as jnp
from jax.experimental import pallas as pl

_ALPHA = 4


def _gather_body(*refs):
    srcs, out = refs[:-1], refs[-1]
    for k, s in enumerate(srcs):
        out[0, :, k] = s[0, :, 0]


def _gather_body_4d(*refs):
    srcs, out = refs[:-1], refs[-1]
    for k, s in enumerate(srcs):
        out[:, k] = s[:, 0]


def kernel(frames):
    temporal_axis = 1 if frames.ndim == 4 else 2
    T = frames.shape[temporal_axis]
    S = T // _ALPHA
    idx = tuple(int(v) for v in np.linspace(0.0, T - 1, S))

    if frames.ndim == 4:
        C, _, H, W = frames.shape

        def _spec(t):
            return pl.BlockSpec((C, 1, H, W), lambda _, _t=t: (0, _t, 0, 0))

        slow = pl.pallas_call(
            _gather_body_4d,
            grid=(1,),
            in_specs=[_spec(t) for t in idx],
            out_specs=pl.BlockSpec((C, S, H, W), lambda _: (0, 0, 0, 0)),
            out_shape=jax.ShapeDtypeStruct((C, S, H, W), frames.dtype),
        )(*([frames] * S))
        return (slow, frames)

    B, C, _, H, W = frames.shape

    def _spec5(t):
        return pl.BlockSpec((1, C, 1, H, W), lambda b, _t=t: (b, 0, _t, 0, 0))

    slow = pl.pallas_call(
        _gather_body,
        grid=(B,),
        in_specs=[_spec5(t) for t in idx],
        out_specs=pl.BlockSpec((1, C, S, H, W), lambda b: (b, 0, 0, 0, 0)),
        out_shape=jax.ShapeDtypeStruct((B, C, S, H, W), frames.dtype),
    )(*([frames] * S))
    return (slow, frames)
